# Initial kernel scaffold; baseline (speedup 1.0000x reference)
import jax
import jax.numpy as jnp
from jax.experimental import pallas as pl


def kernel(inputs, W1, b1, W2, b2, W3, b3):
    raise NotImplementedError("write your pallas kernel here")


# speedup vs baseline: 1.2837x; 1.2837x over previous
import jax
import jax.numpy as jnp
from jax.experimental import pallas as pl
from jax.experimental.pallas import tpu as pltpu

_K = 8
_EPS = 1e-12


def _bf(x):
    return x.astype(jnp.bfloat16)


def _dot(a, b):
    return jnp.dot(a, b, preferred_element_type=jnp.float32)


def _fwd_kernel(xb_ref, w1tb_ref, w2tb_ref, w1t_ref, b1_ref, b2_ref,
                z1_ref, a2_ref, n1_ref):
    xb = xb_ref[...]
    z1 = _dot(xb, w1tb_ref[...]) + b1_ref[...]
    z1_ref[...] = z1[:, None, :]
    a1 = jnp.where(z1 > 0.0, b1_ref[...], 0.0)
    a2 = _dot(_bf(a1), w2tb_ref[...]) + b2_ref[...]
    a2_ref[...] = a2[:, None, :]
    w1t = w1t_ref[...]
    n1_ref[...] = jnp.sqrt(jnp.sum(w1t * w1t, axis=0, keepdims=True))


def _dist_kernel(xb_ref, w1tb_ref, w2tb_ref, w3tb_ref, z1_ref, a2_ref,
                 n1_ref, b3_ref, out_ref):
    z1 = z1_ref[...].reshape(1, -1)
    a1tb = jnp.where(z1 > 0.0, w1tb_ref[...], jnp.bfloat16(0))
    v2t = _dot(a1tb, w2tb_ref[...])
    n2 = jnp.sqrt(jnp.sum(v2t * v2t, axis=0, keepdims=True))
    xb = xb_ref[...].reshape(1, -1)
    v2tb = _bf(v2t)
    z2 = _dot(xb, v2tb) + a2_ref[...].reshape(1, -1)
    m2 = z2 > 0.0
    v2mtb = jnp.where(m2, v2tb, jnp.bfloat16(0))
    v3t = _dot(v2mtb, w3tb_ref[...])
    n3 = jnp.sqrt(jnp.sum(v3t * v3t, axis=0, keepdims=True))
    a2m = jnp.where(m2, a2_ref[...].reshape(1, -1), 0.0)
    a3 = _dot(_bf(a2m), w3tb_ref[...]) + b3_ref[...]
    z3 = _dot(xb, _bf(v3t)) + a3
    d1 = jnp.abs(z1) / (n1_ref[...] + _EPS)
    d2 = jnp.abs(z2) / (n2 + _EPS)
    d3 = jnp.abs(z3) / (n3 + _EPS)
    out_ref[...] = jnp.concatenate([d1, d2, d3], axis=1)[:, None, :]


def _select_kernel(d_ref, out_ref):
    v = d_ref[...].reshape(d_ref.shape[0], d_ref.shape[2])
    b, n = v.shape
    idx = jax.lax.broadcasted_iota(jnp.int32, (b, n), 1)
    acc = jnp.zeros((), dtype=jnp.float32)
    for _ in range(_K):
        row_min = jnp.min(v, axis=1, keepdims=True)
        acc = acc + jnp.sum(row_min)
        is_min = v == row_min
        min_idx = jnp.min(jnp.where(is_min, idx, n), axis=1, keepdims=True)
        v = jnp.where(idx == min_idx, jnp.float32(jnp.inf), v)
    out_ref[...] = jnp.broadcast_to(acc / _K, (1, 1))


@jax.jit
def kernel(inputs, W1, b1, W2, b2, W3, b3):
    B, d_in = inputs.shape
    h1 = W1.shape[0]
    h2 = W2.shape[0]
    h3 = W3.shape[0]
    w1t = W1.T
    w1tb = w1t.astype(jnp.bfloat16)
    w2tb = W2.T.astype(jnp.bfloat16)
    w3tb = W3.T.astype(jnp.bfloat16)
    xb = inputs.astype(jnp.bfloat16)
    b1r = b1[None, :]
    b2r = b2[None, :]
    b3r = b3[None, :]

    z1, a2, n1 = pl.pallas_call(
        _fwd_kernel,
        out_shape=[
            jax.ShapeDtypeStruct((B, 1, h1), jnp.float32),
            jax.ShapeDtypeStruct((B, 1, h2), jnp.float32),
            jax.ShapeDtypeStruct((1, h1), jnp.float32),
        ],
    )(xb, w1tb, w2tb, w1t, b1r, b2r)

    n_total = h1 + h2 + h3
    xb3 = xb[:, None, :]
    dists = pl.pallas_call(
        _dist_kernel,
        grid=(B,),
        in_specs=[
            pl.BlockSpec((1, 1, d_in), lambda b: (b, 0, 0)),
            pl.BlockSpec((d_in, h1), lambda b: (0, 0)),
            pl.BlockSpec((h1, h2), lambda b: (0, 0)),
            pl.BlockSpec((h2, h3), lambda b: (0, 0)),
            pl.BlockSpec((1, 1, h1), lambda b: (b, 0, 0)),
            pl.BlockSpec((1, 1, h2), lambda b: (b, 0, 0)),
            pl.BlockSpec((1, h1), lambda b: (0, 0)),
            pl.BlockSpec((1, h3), lambda b: (0, 0)),
        ],
        out_specs=pl.BlockSpec((1, 1, n_total), lambda b: (b, 0, 0)),
        out_shape=jax.ShapeDtypeStruct((B, 1, n_total), jnp.float32),
    )(xb3, w1tb, w2tb, w3tb, z1, a2, n1, b3r)

    total = pl.pallas_call(
        _select_kernel,
        out_shape=jax.ShapeDtypeStruct((1, 1), jnp.float32),
    )(dists)

    border_dist_sum = total[0, 0]
    fct_dist_sum = jnp.zeros((), dtype=inputs.dtype)
    return (border_dist_sum, fct_dist_sum)


# speedup vs baseline: 1.5108x; 1.1769x over previous
import jax
import jax.numpy as jnp
from jax.experimental import pallas as pl
from jax.experimental.pallas import tpu as pltpu

_K = 8
_EPS = 1e-12
_BT = 2


def _bf(x):
    return x.astype(jnp.bfloat16)


def _dot(a, b):
    return jnp.dot(a, b, preferred_element_type=jnp.float32)


def _fwd_kernel(xb_ref, w1tb_ref, w2tb_ref, w1t_ref, b1_ref, b2_ref,
                z1_ref, a2_ref, n1_ref):
    xb = xb_ref[...]
    z1 = _dot(xb, w1tb_ref[...]) + b1_ref[...]
    z1_ref[...] = z1[:, None, :]
    a1 = jnp.where(z1 > 0.0, b1_ref[...], 0.0)
    a2 = _dot(_bf(a1), w2tb_ref[...]) + b2_ref[...]
    a2_ref[...] = a2[:, None, :]
    w1t = w1t_ref[...]
    n1_ref[...] = jnp.sqrt(jnp.sum(w1t * w1t, axis=0, keepdims=True))


def _dist_kernel(xb_ref, w1tb_ref, w2tb_ref, w3tb_ref, z1_ref, a2_ref,
                 n1_ref, b3_ref, out_ref):
    bt = z1_ref.shape[0]
    d = w1tb_ref.shape[0]
    w1tb = w1tb_ref[...]
    bf0 = jnp.bfloat16(0)
    z1_rows = [z1_ref[i] for i in range(bt)]
    a1tb = jnp.concatenate(
        [jnp.where(z1_rows[i] > 0.0, w1tb, bf0) for i in range(bt)],
        axis=0)
    v2t_all = _dot(a1tb, w2tb_ref[...])
    v2tb_all = _bf(v2t_all)
    n2_rows, z2_rows, m2_rows = [], [], []
    for i in range(bt):
        v2t_i = v2t_all[i * d:(i + 1) * d]
        n2_rows.append(jnp.sqrt(jnp.sum(v2t_i * v2t_i, axis=0, keepdims=True)))
        z2 = _dot(xb_ref[i], v2tb_all[i * d:(i + 1) * d]) + a2_ref[i]
        z2_rows.append(z2)
        m2_rows.append(z2 > 0.0)
    v2mtb = jnp.concatenate(
        [jnp.where(m2_rows[i], v2tb_all[i * d:(i + 1) * d], bf0)
         for i in range(bt)], axis=0)
    v3t_all = _dot(v2mtb, w3tb_ref[...])
    v3tb_all = _bf(v3t_all)
    dist_rows = []
    for i in range(bt):
        v3t_i = v3t_all[i * d:(i + 1) * d]
        n3 = jnp.sqrt(jnp.sum(v3t_i * v3t_i, axis=0, keepdims=True))
        a2m = jnp.where(m2_rows[i], a2_ref[i], 0.0)
        a3 = _dot(_bf(a2m), w3tb_ref[...]) + b3_ref[...]
        z3 = _dot(xb_ref[i], v3tb_all[i * d:(i + 1) * d]) + a3
        d1 = jnp.abs(z1_rows[i]) / (n1_ref[...] + _EPS)
        d2 = jnp.abs(z2_rows[i]) / (n2_rows[i] + _EPS)
        d3 = jnp.abs(z3) / (n3 + _EPS)
        dist_rows.append(jnp.concatenate([d1, d2, d3], axis=1))
    out_ref[...] = jnp.concatenate(dist_rows, axis=0)[:, None, :]


def _select_kernel(d_ref, out_ref):
    v = d_ref[...].reshape(d_ref.shape[0], d_ref.shape[2])
    b, n = v.shape
    idx = jax.lax.broadcasted_iota(jnp.int32, (b, n), 1)
    acc = jnp.zeros((), dtype=jnp.float32)
    for _ in range(_K):
        row_min = jnp.min(v, axis=1, keepdims=True)
        acc = acc + jnp.sum(row_min)
        is_min = v == row_min
        min_idx = jnp.min(jnp.where(is_min, idx, n), axis=1, keepdims=True)
        v = jnp.where(idx == min_idx, jnp.float32(jnp.inf), v)
    out_ref[...] = jnp.broadcast_to(acc / _K, (1, 1))


@jax.jit
def kernel(inputs, W1, b1, W2, b2, W3, b3):
    B, d_in = inputs.shape
    h1 = W1.shape[0]
    h2 = W2.shape[0]
    h3 = W3.shape[0]
    w1t = W1.T
    w1tb = w1t.astype(jnp.bfloat16)
    w2tb = W2.T.astype(jnp.bfloat16)
    w3tb = W3.T.astype(jnp.bfloat16)
    xb = inputs.astype(jnp.bfloat16)
    b1r = b1[None, :]
    b2r = b2[None, :]
    b3r = b3[None, :]

    z1, a2, n1 = pl.pallas_call(
        _fwd_kernel,
        out_shape=[
            jax.ShapeDtypeStruct((B, 1, h1), jnp.float32),
            jax.ShapeDtypeStruct((B, 1, h2), jnp.float32),
            jax.ShapeDtypeStruct((1, h1), jnp.float32),
        ],
    )(xb, w1tb, w2tb, w1t, b1r, b2r)

    n_total = h1 + h2 + h3
    xb3 = xb[:, None, :]
    bt = _BT
    dists = pl.pallas_call(
        _dist_kernel,
        grid=(B // bt,),
        in_specs=[
            pl.BlockSpec((bt, 1, d_in), lambda b: (b, 0, 0)),
            pl.BlockSpec((d_in, h1), lambda b: (0, 0)),
            pl.BlockSpec((h1, h2), lambda b: (0, 0)),
            pl.BlockSpec((h2, h3), lambda b: (0, 0)),
            pl.BlockSpec((bt, 1, h1), lambda b: (b, 0, 0)),
            pl.BlockSpec((bt, 1, h2), lambda b: (b, 0, 0)),
            pl.BlockSpec((1, h1), lambda b: (0, 0)),
            pl.BlockSpec((1, h3), lambda b: (0, 0)),
        ],
        out_specs=pl.BlockSpec((bt, 1, n_total), lambda b: (b, 0, 0)),
        out_shape=jax.ShapeDtypeStruct((B, 1, n_total), jnp.float32),
    )(xb3, w1tb, w2tb, w3tb, z1, a2, n1, b3r)

    total = pl.pallas_call(
        _select_kernel,
        out_shape=jax.ShapeDtypeStruct((1, 1), jnp.float32),
    )(dists)

    border_dist_sum = total[0, 0]
    fct_dist_sum = jnp.zeros((), dtype=inputs.dtype)
    return (border_dist_sum, fct_dist_sum)


# speedup vs baseline: 1.5894x; 1.0520x over previous
import jax
import jax.numpy as jnp
from jax.experimental import pallas as pl
from jax.experimental.pallas import tpu as pltpu

_K = 8
_EPS = 1e-12
_BT = 4


def _bf(x):
    return x.astype(jnp.bfloat16)


def _dot(a, b):
    return jnp.dot(a, b, preferred_element_type=jnp.float32)


def _fwd_kernel(xb_ref, w1tb_ref, w2tb_ref, w1t_ref, b1_ref, b2_ref,
                z1_ref, a2_ref, n1_ref):
    xb = xb_ref[...]
    z1 = _dot(xb, w1tb_ref[...]) + b1_ref[...]
    z1_ref[...] = z1[:, None, :]
    a1 = jnp.where(z1 > 0.0, b1_ref[...], 0.0)
    a2 = _dot(_bf(a1), w2tb_ref[...]) + b2_ref[...]
    a2_ref[...] = a2[:, None, :]
    w1t = w1t_ref[...]
    n1_ref[...] = jnp.sqrt(jnp.sum(w1t * w1t, axis=0, keepdims=True))


def _dist_kernel(xb_ref, w1tb_ref, w2tb_ref, w3tb_ref, z1_ref, a2_ref,
                 n1_ref, b3_ref, out_ref):
    bt = z1_ref.shape[0]
    d = w1tb_ref.shape[0]
    w1tb = w1tb_ref[...]
    bf0 = jnp.bfloat16(0)
    z1_rows = [z1_ref[i] for i in range(bt)]
    a1tb = jnp.concatenate(
        [jnp.where(z1_rows[i] > 0.0, w1tb, bf0) for i in range(bt)],
        axis=0)
    v2t_all = _dot(a1tb, w2tb_ref[...])
    v2tb_all = _bf(v2t_all)
    n2_rows, z2_rows, m2_rows = [], [], []
    for i in range(bt):
        v2t_i = v2t_all[i * d:(i + 1) * d]
        n2_rows.append(jnp.sqrt(jnp.sum(v2t_i * v2t_i, axis=0, keepdims=True)))
        z2 = _dot(xb_ref[i], v2tb_all[i * d:(i + 1) * d]) + a2_ref[i]
        z2_rows.append(z2)
        m2_rows.append(z2 > 0.0)
    v2mtb = jnp.concatenate(
        [jnp.where(m2_rows[i], v2tb_all[i * d:(i + 1) * d], bf0)
         for i in range(bt)], axis=0)
    v3t_all = _dot(v2mtb, w3tb_ref[...])
    v3tb_all = _bf(v3t_all)
    dist_rows = []
    for i in range(bt):
        v3t_i = v3t_all[i * d:(i + 1) * d]
        n3 = jnp.sqrt(jnp.sum(v3t_i * v3t_i, axis=0, keepdims=True))
        a2m = jnp.where(m2_rows[i], a2_ref[i], 0.0)
        a3 = _dot(_bf(a2m), w3tb_ref[...]) + b3_ref[...]
        z3 = _dot(xb_ref[i], v3tb_all[i * d:(i + 1) * d]) + a3
        d1 = jnp.abs(z1_rows[i]) / (n1_ref[...] + _EPS)
        d2 = jnp.abs(z2_rows[i]) / (n2_rows[i] + _EPS)
        d3 = jnp.abs(z3) / (n3 + _EPS)
        dist_rows.append(jnp.concatenate([d1, d2, d3], axis=1))
    out_ref[...] = jnp.concatenate(dist_rows, axis=0)[:, None, :]


def _select_kernel(d_ref, out_ref):
    v = d_ref[...].reshape(d_ref.shape[0], d_ref.shape[2])
    b, n = v.shape
    idx = jax.lax.broadcasted_iota(jnp.int32, (b, n), 1)
    acc = jnp.zeros((), dtype=jnp.float32)
    for _ in range(_K):
        row_min = jnp.min(v, axis=1, keepdims=True)
        acc = acc + jnp.sum(row_min)
        is_min = v == row_min
        min_idx = jnp.min(jnp.where(is_min, idx, n), axis=1, keepdims=True)
        v = jnp.where(idx == min_idx, jnp.float32(jnp.inf), v)
    out_ref[...] = jnp.broadcast_to(acc / _K, (1, 1))


@jax.jit
def kernel(inputs, W1, b1, W2, b2, W3, b3):
    B, d_in = inputs.shape
    h1 = W1.shape[0]
    h2 = W2.shape[0]
    h3 = W3.shape[0]
    w1t = W1.T
    w1tb = w1t.astype(jnp.bfloat16)
    w2tb = W2.T.astype(jnp.bfloat16)
    w3tb = W3.T.astype(jnp.bfloat16)
    xb = inputs.astype(jnp.bfloat16)
    b1r = b1[None, :]
    b2r = b2[None, :]
    b3r = b3[None, :]

    z1, a2, n1 = pl.pallas_call(
        _fwd_kernel,
        out_shape=[
            jax.ShapeDtypeStruct((B, 1, h1), jnp.float32),
            jax.ShapeDtypeStruct((B, 1, h2), jnp.float32),
            jax.ShapeDtypeStruct((1, h1), jnp.float32),
        ],
    )(xb, w1tb, w2tb, w1t, b1r, b2r)

    n_total = h1 + h2 + h3
    xb3 = xb[:, None, :]
    bt = _BT
    dists = pl.pallas_call(
        _dist_kernel,
        grid=(B // bt,),
        in_specs=[
            pl.BlockSpec((bt, 1, d_in), lambda b: (b, 0, 0)),
            pl.BlockSpec((d_in, h1), lambda b: (0, 0)),
            pl.BlockSpec((h1, h2), lambda b: (0, 0)),
            pl.BlockSpec((h2, h3), lambda b: (0, 0)),
            pl.BlockSpec((bt, 1, h1), lambda b: (b, 0, 0)),
            pl.BlockSpec((bt, 1, h2), lambda b: (b, 0, 0)),
            pl.BlockSpec((1, h1), lambda b: (0, 0)),
            pl.BlockSpec((1, h3), lambda b: (0, 0)),
        ],
        out_specs=pl.BlockSpec((bt, 1, n_total), lambda b: (b, 0, 0)),
        out_shape=jax.ShapeDtypeStruct((B, 1, n_total), jnp.float32),
    )(xb3, w1tb, w2tb, w3tb, z1, a2, n1, b3r)

    total = pl.pallas_call(
        _select_kernel,
        out_shape=jax.ShapeDtypeStruct((1, 1), jnp.float32),
    )(dists)

    border_dist_sum = total[0, 0]
    fct_dist_sum = jnp.zeros((), dtype=inputs.dtype)
    return (border_dist_sum, fct_dist_sum)


# speedup vs baseline: 1.6103x; 1.0132x over previous
import jax
import jax.numpy as jnp
from jax.experimental import pallas as pl
from jax.experimental.pallas import tpu as pltpu

_K = 8
_EPS = 1e-12
_BT = 8


def _bf(x):
    return x.astype(jnp.bfloat16)


def _dot(a, b):
    return jnp.dot(a, b, preferred_element_type=jnp.float32)


def _fwd_kernel(xb_ref, w1tb_ref, w2tb_ref, w1t_ref, b1_ref, b2_ref,
                z1_ref, a2_ref, n1_ref):
    xb = xb_ref[...]
    z1 = _dot(xb, w1tb_ref[...]) + b1_ref[...]
    z1_ref[...] = z1[:, None, :]
    a1 = jnp.where(z1 > 0.0, b1_ref[...], 0.0)
    a2 = _dot(_bf(a1), w2tb_ref[...]) + b2_ref[...]
    a2_ref[...] = a2[:, None, :]
    w1t = w1t_ref[...]
    n1_ref[...] = jnp.sqrt(jnp.sum(w1t * w1t, axis=0, keepdims=True))


def _dist_kernel(xb_ref, w1tb_ref, w2tb_ref, w3tb_ref, z1_ref, a2_ref,
                 n1_ref, b3_ref, out_ref):
    bt = z1_ref.shape[0]
    d = w1tb_ref.shape[0]
    w1tb = w1tb_ref[...]
    bf0 = jnp.bfloat16(0)
    z1_rows = [z1_ref[i] for i in range(bt)]
    a1tb = jnp.concatenate(
        [jnp.where(z1_rows[i] > 0.0, w1tb, bf0) for i in range(bt)],
        axis=0)
    v2t_all = _dot(a1tb, w2tb_ref[...])
    v2tb_all = _bf(v2t_all)
    n2_rows, z2_rows, m2_rows = [], [], []
    for i in range(bt):
        v2t_i = v2t_all[i * d:(i + 1) * d]
        n2_rows.append(jnp.sqrt(jnp.sum(v2t_i * v2t_i, axis=0, keepdims=True)))
        z2 = _dot(xb_ref[i], v2tb_all[i * d:(i + 1) * d]) + a2_ref[i]
        z2_rows.append(z2)
        m2_rows.append(z2 > 0.0)
    v2mtb = jnp.concatenate(
        [jnp.where(m2_rows[i], v2tb_all[i * d:(i + 1) * d], bf0)
         for i in range(bt)], axis=0)
    v3t_all = _dot(v2mtb, w3tb_ref[...])
    v3tb_all = _bf(v3t_all)
    dist_rows = []
    for i in range(bt):
        v3t_i = v3t_all[i * d:(i + 1) * d]
        n3 = jnp.sqrt(jnp.sum(v3t_i * v3t_i, axis=0, keepdims=True))
        a2m = jnp.where(m2_rows[i], a2_ref[i], 0.0)
        a3 = _dot(_bf(a2m), w3tb_ref[...]) + b3_ref[...]
        z3 = _dot(xb_ref[i], v3tb_all[i * d:(i + 1) * d]) + a3
        d1 = jnp.abs(z1_rows[i]) / (n1_ref[...] + _EPS)
        d2 = jnp.abs(z2_rows[i]) / (n2_rows[i] + _EPS)
        d3 = jnp.abs(z3) / (n3 + _EPS)
        dist_rows.append(jnp.concatenate([d1, d2, d3], axis=1))
    out_ref[...] = jnp.concatenate(dist_rows, axis=0)[:, None, :]


def _select_kernel(d_ref, out_ref):
    v = d_ref[...].reshape(d_ref.shape[0], d_ref.shape[2])
    b, n = v.shape
    idx = jax.lax.broadcasted_iota(jnp.int32, (b, n), 1)
    acc = jnp.zeros((), dtype=jnp.float32)
    for _ in range(_K):
        row_min = jnp.min(v, axis=1, keepdims=True)
        acc = acc + jnp.sum(row_min)
        is_min = v == row_min
        min_idx = jnp.min(jnp.where(is_min, idx, n), axis=1, keepdims=True)
        v = jnp.where(idx == min_idx, jnp.float32(jnp.inf), v)
    out_ref[...] = jnp.broadcast_to(acc / _K, (1, 1))


@jax.jit
def kernel(inputs, W1, b1, W2, b2, W3, b3):
    B, d_in = inputs.shape
    h1 = W1.shape[0]
    h2 = W2.shape[0]
    h3 = W3.shape[0]
    w1t = W1.T
    w1tb = w1t.astype(jnp.bfloat16)
    w2tb = W2.T.astype(jnp.bfloat16)
    w3tb = W3.T.astype(jnp.bfloat16)
    xb = inputs.astype(jnp.bfloat16)
    b1r = b1[None, :]
    b2r = b2[None, :]
    b3r = b3[None, :]

    z1, a2, n1 = pl.pallas_call(
        _fwd_kernel,
        out_shape=[
            jax.ShapeDtypeStruct((B, 1, h1), jnp.float32),
            jax.ShapeDtypeStruct((B, 1, h2), jnp.float32),
            jax.ShapeDtypeStruct((1, h1), jnp.float32),
        ],
    )(xb, w1tb, w2tb, w1t, b1r, b2r)

    n_total = h1 + h2 + h3
    xb3 = xb[:, None, :]
    bt = _BT
    dists = pl.pallas_call(
        _dist_kernel,
        grid=(B // bt,),
        in_specs=[
            pl.BlockSpec((bt, 1, d_in), lambda b: (b, 0, 0)),
            pl.BlockSpec((d_in, h1), lambda b: (0, 0)),
            pl.BlockSpec((h1, h2), lambda b: (0, 0)),
            pl.BlockSpec((h2, h3), lambda b: (0, 0)),
            pl.BlockSpec((bt, 1, h1), lambda b: (b, 0, 0)),
            pl.BlockSpec((bt, 1, h2), lambda b: (b, 0, 0)),
            pl.BlockSpec((1, h1), lambda b: (0, 0)),
            pl.BlockSpec((1, h3), lambda b: (0, 0)),
        ],
        out_specs=pl.BlockSpec((bt, 1, n_total), lambda b: (b, 0, 0)),
        out_shape=jax.ShapeDtypeStruct((B, 1, n_total), jnp.float32),
    )(xb3, w1tb, w2tb, w3tb, z1, a2, n1, b3r)

    total = pl.pallas_call(
        _select_kernel,
        out_shape=jax.ShapeDtypeStruct((1, 1), jnp.float32),
    )(dists)

    border_dist_sum = total[0, 0]
    fct_dist_sum = jnp.zeros((), dtype=inputs.dtype)
    return (border_dist_sum, fct_dist_sum)
